# Initial kernel scaffold; baseline (speedup 1.0000x reference)
import jax
import jax.numpy as jnp
from jax.experimental import pallas as pl


def kernel(z1, z2, atom_weight, pair_ids, onehot_table):
    raise NotImplementedError("write your pallas kernel here")


# speedup vs baseline: 10.0055x; 10.0055x over previous
import functools

import jax
import jax.numpy as jnp
from jax import lax
from jax.experimental import pallas as pl
from jax.experimental.pallas import tpu as pltpu
from jax.experimental.pallas import tpu_sc as plsc

_NC = 2
_NS = 16
_NW = _NC * _NS
_LANES = 16
_CH_G = 13
_CH = _CH_G * _LANES


@functools.cache
def _build(E, A, n, C):
    per_w = E // _NW
    assert per_w * _NW == E
    nch = per_w // _CH
    tail = per_w - nch * _CH
    fb = _CH * C
    assert nch >= 4 and nch % 2 == 0
    assert 0 < tail < _LANES
    assert per_w % 8 == 0 and _CH % 8 == 0 and fb % 8 == 0
    assert (per_w * C) % 8 == 0 and (tail * C) % 8 == 0
    eb = ((tail * C + _LANES - 1) // _LANES) * _LANES

    mesh = plsc.VectorSubcoreMesh(core_axis_name="c", subcore_axis_name="s")

    @functools.partial(
        pl.kernel,
        out_type=jax.ShapeDtypeStruct((E * C,), jnp.float32),
        mesh=mesh,
        compiler_params=pltpu.CompilerParams(needs_layout_passes=False),
        scratch_types=[
            pltpu.VMEM((fb,), jnp.float32),
            pltpu.VMEM((fb,), jnp.float32),
            pltpu.VMEM((per_w,), jnp.int32),
            pltpu.VMEM((per_w,), jnp.int32),
            pltpu.VMEM((_CH,), jnp.int32),
            pltpu.VMEM((_CH,), jnp.int32),
            pltpu.VMEM((A,), jnp.int32),
            pltpu.VMEM((n * n,), jnp.int32),
            pltpu.VMEM((eb,), jnp.float32),
            pltpu.VMEM((_LANES,), jnp.int32),
            pltpu.VMEM((_LANES,), jnp.int32),
            pltpu.SemaphoreType.DMA,
            pltpu.SemaphoreType.DMA,
        ],
    )
    def sc_call(z1_hbm, z2_hbm, aw_hbm, pi_hbm, out_hbm,
                fb0, fb1, z1v, z2v, ix0, ix1, aw_v, pi_v, ebuf, t1, t2,
                sem0, sem1):
        wid = lax.axis_index("s") * _NC + lax.axis_index("c")
        base = wid * per_w
        obase = base * C

        pltpu.sync_copy(aw_hbm, aw_v)
        pltpu.sync_copy(pi_hbm, pi_v)
        pltpu.sync_copy(z1_hbm.at[pl.ds(base, per_w)], z1v)
        pltpu.sync_copy(z2_hbm.at[pl.ds(base, per_w)], z2v)

        zf = jnp.zeros((_LANES,), jnp.float32)
        onesf = jnp.ones((_LANES,), jnp.float32)
        lanes = lax.broadcasted_iota(jnp.int32, (_LANES,), 0)
        lanes_c = lanes * C

        def zero_body(i, carry):
            for j in range(_CH_G):
                off = (i * _CH_G + j) * _LANES
                fb0[pl.ds(off, _LANES)] = zf
                fb1[pl.ds(off, _LANES)] = zf
            return carry
        lax.fori_loop(0, fb // (_LANES * _CH_G), zero_body, 0)

        def zero_eb(i, carry):
            ebuf[pl.ds(i * _LANES, _LANES)] = zf
            return carry
        lax.fori_loop(0, eb // _LANES, zero_eb, 0)

        fbs = (fb0, fb1)
        ixs = (ix0, ix1)
        sems = (sem0, sem1)

        def pairtype(a_raw, b_raw):
            ia = plsc.load_gather(aw_v, [a_raw - 1])
            ib = plsc.load_gather(aw_v, [b_raw - 1])
            return plsc.load_gather(pi_v, [ia * n + ib])

        def compute_chunk(c, buf, ix):
            zoff = c * _CH
            for g in range(_CH_G):
                pt = pairtype(z1v[pl.ds(zoff + g * _LANES, _LANES)],
                              z2v[pl.ds(zoff + g * _LANES, _LANES)])
                flat = lanes_c + (g * _LANES * C) + pt
                plsc.store_scatter(buf, [flat], onesf)
                ix[pl.ds(g * _LANES, _LANES)] = flat

        def out_copy(c, buf, sem):
            return pltpu.make_async_copy(
                buf, out_hbm.at[pl.ds(obase + c * fb, fb)], sem)

        def rezero(buf, ix):
            for g in range(_CH_G):
                plsc.store_scatter(buf, [ix[pl.ds(g * _LANES, _LANES)]], zf)

        for b in range(2):
            compute_chunk(b, fbs[b], ixs[b])
            out_copy(b, fbs[b], sems[b]).start()

        def loop_body(t, carry):
            for b in range(2):
                c = 2 * t + b
                out_copy(c - 2, fbs[b], sems[b]).wait()
                rezero(fbs[b], ixs[b])
                compute_chunk(c, fbs[b], ixs[b])
                out_copy(c, fbs[b], sems[b]).start()
            return carry
        lax.fori_loop(1, nch // 2, loop_body, 0)

        for b in range(2):
            out_copy(nch - 2 + b, fbs[b], sems[b]).wait()

        ones_i = jnp.ones((_LANES,), jnp.int32)
        t1[pl.ds(0, _LANES)] = ones_i
        t2[pl.ds(0, _LANES)] = ones_i
        toff = base + nch * _CH
        pltpu.sync_copy(z1_hbm.at[pl.ds(toff, tail)], t1.at[pl.ds(0, tail)])
        pltpu.sync_copy(z2_hbm.at[pl.ds(toff, tail)], t2.at[pl.ds(0, tail)])
        pt = pairtype(t1[pl.ds(0, _LANES)], t2[pl.ds(0, _LANES)])
        plsc.store_scatter(ebuf, [lanes_c + pt], onesf, mask=lanes < tail)
        pltpu.sync_copy(ebuf.at[pl.ds(0, tail * C)],
                        out_hbm.at[pl.ds(obase + nch * fb, tail * C)])

    return sc_call


def kernel(z1, z2, atom_weight, pair_ids, onehot_table):
    E = z1.shape[0]
    A = atom_weight.shape[0]
    n = pair_ids.shape[0]
    C = onehot_table.shape[1]
    out_flat = _build(E, A, n, C)(z1, z2, atom_weight, pair_ids.reshape(-1))
    return out_flat.reshape(E, C)


# speedup vs baseline: 14.0009x; 1.3993x over previous
import functools

import jax
import jax.numpy as jnp
from jax import lax
from jax.experimental import pallas as pl
from jax.experimental.pallas import tpu as pltpu
from jax.experimental.pallas import tpu_sc as plsc

_NC = 2
_NS = 16
_NW = _NC * _NS
_LANES = 16


@functools.cache
def _build_pairtype(E, A, n):
    per_w = E // _NW
    assert per_w * _NW == E and per_w % 8 == 0
    n_groups = -(-per_w // _LANES)
    n_full = per_w // _LANES
    buf = n_groups * _LANES
    unroll = 8
    assert n_full % unroll == 0

    mesh = plsc.VectorSubcoreMesh(core_axis_name="c", subcore_axis_name="s")

    @functools.partial(
        pl.kernel,
        out_type=jax.ShapeDtypeStruct((E,), jnp.int32),
        mesh=mesh,
        compiler_params=pltpu.CompilerParams(needs_layout_passes=False),
        scratch_types=[
            pltpu.VMEM((buf,), jnp.int32),
            pltpu.VMEM((buf,), jnp.int32),
            pltpu.VMEM((buf,), jnp.int32),
            pltpu.VMEM((A,), jnp.int32),
            pltpu.VMEM((n * n,), jnp.int32),
        ],
    )
    def sc_call(z1_hbm, z2_hbm, aw_hbm, pi_hbm, pt_hbm, z1v, z2v, ptv, aw_v, pi_v):
        wid = lax.axis_index("s") * _NC + lax.axis_index("c")
        base = wid * per_w

        ones_i = jnp.ones((_LANES,), jnp.int32)
        z1v[pl.ds(n_full * _LANES, _LANES)] = ones_i
        z2v[pl.ds(n_full * _LANES, _LANES)] = ones_i

        pltpu.sync_copy(aw_hbm, aw_v)
        pltpu.sync_copy(pi_hbm, pi_v)
        pltpu.sync_copy(z1_hbm.at[pl.ds(base, per_w)], z1v.at[pl.ds(0, per_w)])
        pltpu.sync_copy(z2_hbm.at[pl.ds(base, per_w)], z2v.at[pl.ds(0, per_w)])

        def group(g):
            ia = plsc.load_gather(aw_v, [z1v[pl.ds(g * _LANES, _LANES)] - 1])
            ib = plsc.load_gather(aw_v, [z2v[pl.ds(g * _LANES, _LANES)] - 1])
            ptv[pl.ds(g * _LANES, _LANES)] = plsc.load_gather(pi_v, [ia * n + ib])

        def loop_body(t, carry):
            for j in range(unroll):
                group(t * unroll + j)
            return carry
        lax.fori_loop(0, n_full // unroll, loop_body, 0)
        for g in range(n_full, n_groups):
            group(g)

        pltpu.sync_copy(ptv.at[pl.ds(0, per_w)], pt_hbm.at[pl.ds(base, per_w)])

    return sc_call


@functools.cache
def _build_onehot(E, C):
    assert E % 128 == 0
    rows = E // 128
    pt_rows_pb = 8
    out_rows_pb = pt_rows_pb * 128
    grid = -(-rows // pt_rows_pb)

    def body(pt_ref, out_ref):
        ptt = pt_ref[...].T
        ciota = lax.broadcasted_iota(jnp.int32, (128, C), 1)
        for j in range(pt_rows_pb):
            col = ptt[:, j:j + 1]
            out_ref[pl.ds(j * 128, 128), :] = (col == ciota).astype(jnp.float32)

    return pl.pallas_call(
        body,
        grid=(grid,),
        in_specs=[pl.BlockSpec((pt_rows_pb, 128), lambda i: (i, 0))],
        out_specs=pl.BlockSpec((out_rows_pb, C), lambda i: (i, 0)),
        out_shape=jax.ShapeDtypeStruct((E, C), jnp.float32),
    )


def kernel(z1, z2, atom_weight, pair_ids, onehot_table):
    E = z1.shape[0]
    A = atom_weight.shape[0]
    n = pair_ids.shape[0]
    C = onehot_table.shape[1]
    pt = _build_pairtype(E, A, n)(z1, z2, atom_weight, pair_ids.reshape(-1))
    return _build_onehot(E, C)(pt.reshape(E // 128, 128))


# speedup vs baseline: 42.8221x; 3.0585x over previous
import functools

import jax
import jax.numpy as jnp
from jax import lax
from jax.experimental import pallas as pl
from jax.experimental.pallas import tpu as pltpu
from jax.experimental.pallas import tpu_sc as plsc

_NC = 2
_NS = 16
_NW = _NC * _NS
_LANES = 16


@functools.cache
def _build_pairtype(E, A, n):
    per_w = E // _NW
    assert per_w * _NW == E and per_w % 8 == 0
    n_groups = -(-per_w // _LANES)
    n_full = per_w // _LANES
    buf = n_groups * _LANES
    unroll = 8
    assert n_full % unroll == 0

    mesh = plsc.VectorSubcoreMesh(core_axis_name="c", subcore_axis_name="s")

    @functools.partial(
        pl.kernel,
        out_type=jax.ShapeDtypeStruct((E,), jnp.int32),
        mesh=mesh,
        compiler_params=pltpu.CompilerParams(needs_layout_passes=False),
        scratch_types=[
            pltpu.VMEM((buf,), jnp.int32),
            pltpu.VMEM((buf,), jnp.int32),
            pltpu.VMEM((buf,), jnp.int32),
            pltpu.VMEM((A,), jnp.int32),
            pltpu.VMEM((n * n,), jnp.int32),
        ],
    )
    def sc_call(z1_hbm, z2_hbm, aw_hbm, pi_hbm, pt_hbm, z1v, z2v, ptv, aw_v, pi_v):
        wid = lax.axis_index("s") * _NC + lax.axis_index("c")
        base = wid * per_w

        ones_i = jnp.ones((_LANES,), jnp.int32)
        z1v[pl.ds(n_full * _LANES, _LANES)] = ones_i
        z2v[pl.ds(n_full * _LANES, _LANES)] = ones_i

        pltpu.sync_copy(aw_hbm, aw_v)
        pltpu.sync_copy(pi_hbm, pi_v)
        pltpu.sync_copy(z1_hbm.at[pl.ds(base, per_w)], z1v.at[pl.ds(0, per_w)])
        pltpu.sync_copy(z2_hbm.at[pl.ds(base, per_w)], z2v.at[pl.ds(0, per_w)])

        def group(g):
            ia = plsc.load_gather(aw_v, [z1v[pl.ds(g * _LANES, _LANES)] - 1])
            ib = plsc.load_gather(aw_v, [z2v[pl.ds(g * _LANES, _LANES)] - 1])
            ptv[pl.ds(g * _LANES, _LANES)] = plsc.load_gather(pi_v, [ia * n + ib])

        def loop_body(t, carry):
            for j in range(unroll):
                group(t * unroll + j)
            return carry
        lax.fori_loop(0, n_full // unroll, loop_body, 0)
        for g in range(n_full, n_groups):
            group(g)

        pltpu.sync_copy(ptv.at[pl.ds(0, per_w)], pt_hbm.at[pl.ds(base, per_w)])

    return sc_call


@functools.cache
def _build_onehot_t(E, C):
    assert E % 128 == 0
    rows = E // 128
    pt_rows_pb = 16
    cols_pb = pt_rows_pb * 128
    grid = -(-rows // pt_rows_pb)

    def body(pt_ref, out_ref):
        ciota = lax.broadcasted_iota(jnp.int32, (C, 128), 0)
        for j in range(pt_rows_pb):
            row = pt_ref[pl.ds(j, 1), :]
            out_ref[:, pl.ds(j * 128, 128)] = (row == ciota).astype(jnp.float32)

    return pl.pallas_call(
        body,
        grid=(grid,),
        in_specs=[pl.BlockSpec((pt_rows_pb, 128), lambda i: (i, 0))],
        out_specs=pl.BlockSpec((C, cols_pb), lambda i: (0, i)),
        out_shape=jax.ShapeDtypeStruct((C, E), jnp.float32),
    )


def kernel(z1, z2, atom_weight, pair_ids, onehot_table):
    E = z1.shape[0]
    A = atom_weight.shape[0]
    n = pair_ids.shape[0]
    C = onehot_table.shape[1]
    pt = _build_pairtype(E, A, n)(z1, z2, atom_weight, pair_ids.reshape(-1))
    return _build_onehot_t(E, C)(pt.reshape(E // 128, 128)).T


# speedup vs baseline: 54.9331x; 1.2828x over previous
import functools

import jax
import jax.numpy as jnp
from jax import lax
from jax.experimental import pallas as pl
from jax.experimental.pallas import tpu as pltpu
from jax.experimental.pallas import tpu_sc as plsc

_NC = 2
_NS = 16
_NW = _NC * _NS
_LANES = 16


@functools.cache
def _build_pairtype(E, A, n):
    per_w = E // _NW
    assert per_w * _NW == E and per_w % 8 == 0
    n_groups = -(-per_w // _LANES)
    n_full = per_w // _LANES
    buf = n_groups * _LANES
    unroll = 8
    assert n_full % unroll == 0

    mesh = plsc.VectorSubcoreMesh(core_axis_name="c", subcore_axis_name="s")

    @functools.partial(
        pl.kernel,
        out_type=jax.ShapeDtypeStruct((E,), jnp.int32),
        mesh=mesh,
        compiler_params=pltpu.CompilerParams(needs_layout_passes=False),
        scratch_types=[
            pltpu.VMEM((buf,), jnp.int32),
            pltpu.VMEM((buf,), jnp.int32),
            pltpu.VMEM((buf,), jnp.int32),
            pltpu.VMEM((A,), jnp.int32),
            pltpu.VMEM((n * n,), jnp.int32),
        ],
    )
    def sc_call(z1_hbm, z2_hbm, aw_hbm, pi_hbm, pt_hbm, z1v, z2v, ptv, aw_v, pi_v):
        wid = lax.axis_index("s") * _NC + lax.axis_index("c")
        base = wid * per_w

        ones_i = jnp.ones((_LANES,), jnp.int32)
        z1v[pl.ds(n_full * _LANES, _LANES)] = ones_i
        z2v[pl.ds(n_full * _LANES, _LANES)] = ones_i

        pltpu.sync_copy(aw_hbm, aw_v)
        pltpu.sync_copy(pi_hbm, pi_v)
        pltpu.sync_copy(z1_hbm.at[pl.ds(base, per_w)], z1v.at[pl.ds(0, per_w)])
        pltpu.sync_copy(z2_hbm.at[pl.ds(base, per_w)], z2v.at[pl.ds(0, per_w)])

        def group(g):
            ia = plsc.load_gather(aw_v, [z1v[pl.ds(g * _LANES, _LANES)] - 1])
            ib = plsc.load_gather(aw_v, [z2v[pl.ds(g * _LANES, _LANES)] - 1])
            ptv[pl.ds(g * _LANES, _LANES)] = plsc.load_gather(pi_v, [ia * n + ib])

        def loop_body(t, carry):
            for j in range(unroll):
                group(t * unroll + j)
            return carry
        lax.fori_loop(0, n_full // unroll, loop_body, 0)
        for g in range(n_full, n_groups):
            group(g)

        pltpu.sync_copy(ptv.at[pl.ds(0, per_w)], pt_hbm.at[pl.ds(base, per_w)])

    return sc_call


@functools.cache
def _build_onehot_t(E, C):
    assert E % 128 == 0
    rows = E // 128
    pt_rows_pb = 32
    cols_pb = pt_rows_pb * 128
    grid = -(-rows // pt_rows_pb)

    def body(pt_ref, out_ref):
        ciota = lax.broadcasted_iota(jnp.int32, (C, 128), 0)
        for j in range(pt_rows_pb):
            row = pt_ref[pl.ds(j, 1), :]
            out_ref[:, pl.ds(j * 128, 128)] = (row == ciota).astype(jnp.float32)

    return pl.pallas_call(
        body,
        grid=(grid,),
        in_specs=[pl.BlockSpec((pt_rows_pb, 128), lambda i: (i, 0))],
        out_specs=pl.BlockSpec((C, cols_pb), lambda i: (0, i)),
        out_shape=jax.ShapeDtypeStruct((C, E), jnp.float32),
    )


def kernel(z1, z2, atom_weight, pair_ids, onehot_table):
    E = z1.shape[0]
    A = atom_weight.shape[0]
    n = pair_ids.shape[0]
    C = onehot_table.shape[1]
    pt = _build_pairtype(E, A, n)(z1, z2, atom_weight, pair_ids.reshape(-1))
    return _build_onehot_t(E, C)(pt.reshape(E // 128, 128)).T


# speedup vs baseline: 61.2254x; 1.1145x over previous
import functools

import jax
import jax.numpy as jnp
from jax import lax
from jax.experimental import pallas as pl
from jax.experimental.pallas import tpu as pltpu
from jax.experimental.pallas import tpu_sc as plsc

_NC = 2
_NS = 16
_NW = _NC * _NS
_LANES = 16


@functools.cache
def _build_pairtype(E, A, n):
    per_w = E // _NW
    assert per_w * _NW == E and per_w % 8 == 0
    n_groups = -(-per_w // _LANES)
    n_full = per_w // _LANES
    buf = n_groups * _LANES
    unroll = 8
    assert n_full % unroll == 0

    mesh = plsc.VectorSubcoreMesh(core_axis_name="c", subcore_axis_name="s")

    @functools.partial(
        pl.kernel,
        out_type=jax.ShapeDtypeStruct((E,), jnp.int32),
        mesh=mesh,
        compiler_params=pltpu.CompilerParams(needs_layout_passes=False),
        scratch_types=[
            pltpu.VMEM((buf,), jnp.int32),
            pltpu.VMEM((buf,), jnp.int32),
            pltpu.VMEM((buf,), jnp.int32),
            pltpu.VMEM((A,), jnp.int32),
            pltpu.VMEM((n * n,), jnp.int32),
        ],
    )
    def sc_call(z1_hbm, z2_hbm, aw_hbm, pi_hbm, pt_hbm, z1v, z2v, ptv, aw_v, pi_v):
        wid = lax.axis_index("s") * _NC + lax.axis_index("c")
        base = wid * per_w

        ones_i = jnp.ones((_LANES,), jnp.int32)
        z1v[pl.ds(n_full * _LANES, _LANES)] = ones_i
        z2v[pl.ds(n_full * _LANES, _LANES)] = ones_i

        pltpu.sync_copy(aw_hbm, aw_v)
        pltpu.sync_copy(pi_hbm, pi_v)
        pltpu.sync_copy(z1_hbm.at[pl.ds(base, per_w)], z1v.at[pl.ds(0, per_w)])
        pltpu.sync_copy(z2_hbm.at[pl.ds(base, per_w)], z2v.at[pl.ds(0, per_w)])

        def group(g):
            ia = plsc.load_gather(aw_v, [z1v[pl.ds(g * _LANES, _LANES)] - 1])
            ib = plsc.load_gather(aw_v, [z2v[pl.ds(g * _LANES, _LANES)] - 1])
            ptv[pl.ds(g * _LANES, _LANES)] = plsc.load_gather(pi_v, [ia * n + ib])

        def loop_body(t, carry):
            for j in range(unroll):
                group(t * unroll + j)
            return carry
        lax.fori_loop(0, n_full // unroll, loop_body, 0)
        for g in range(n_full, n_groups):
            group(g)

        pltpu.sync_copy(ptv.at[pl.ds(0, per_w)], pt_hbm.at[pl.ds(base, per_w)])

    return sc_call


@functools.cache
def _build_onehot_t(E, C):
    assert E % 128 == 0
    rows = E // 128
    pt_rows_pb = 64
    cols_pb = pt_rows_pb * 128
    grid = -(-rows // pt_rows_pb)

    def body(pt_ref, out_ref):
        ciota = lax.broadcasted_iota(jnp.int32, (C, 128), 0)
        for j in range(pt_rows_pb):
            row = pt_ref[pl.ds(j, 1), :]
            out_ref[:, pl.ds(j * 128, 128)] = (row == ciota).astype(jnp.float32)

    return pl.pallas_call(
        body,
        grid=(grid,),
        in_specs=[pl.BlockSpec((pt_rows_pb, 128), lambda i: (i, 0))],
        out_specs=pl.BlockSpec((C, cols_pb), lambda i: (0, i)),
        out_shape=jax.ShapeDtypeStruct((C, E), jnp.float32),
    )


def kernel(z1, z2, atom_weight, pair_ids, onehot_table):
    E = z1.shape[0]
    A = atom_weight.shape[0]
    n = pair_ids.shape[0]
    C = onehot_table.shape[1]
    pt = _build_pairtype(E, A, n)(z1, z2, atom_weight, pair_ids.reshape(-1))
    return _build_onehot_t(E, C)(pt.reshape(E // 128, 128)).T
